# baseline (speedup 1.0000x reference)
import functools

import jax
import jax.numpy as jnp
from jax import lax
from jax.experimental import pallas as pl
from jax.experimental.pallas import tpu as pltpu
from jax.experimental.pallas import tpu_sc as plsc

_N = 10000
_E = 320000
_D = 128
_H = 128
_C = 2
_G = 64

_NC = 2
_NS = 16
_L = 16
_NW = _NC * _NS
_EPW = _E // _NW
_K = 80
_NCH = _EPW // _K
_RPT = 624
_REM = _N - _NS * _RPT
_REM0 = _NS * _RPT


def _weighted_segsum(x, edges3, w3, zeros_tile):
    mesh = plsc.VectorSubcoreMesh(core_axis_name="c", subcore_axis_name="s")

    @functools.partial(
        pl.kernel,
        mesh=mesh,
        out_type=jax.ShapeDtypeStruct((_NC, _N, _D), jnp.float32),
        scratch_types=[
            pltpu.VMEM((2, _K), jnp.int32),
            pltpu.VMEM((2, _K), jnp.int32),
            pltpu.VMEM((2, _K), jnp.int32),
            pltpu.VMEM((1, _K), jnp.float32),
            pltpu.VMEM((1, _K), jnp.float32),
            pltpu.VMEM((_K, _D), jnp.float32),
            pltpu.VMEM((_K, _D), jnp.float32),
            pltpu.VMEM((_K, _D), jnp.float32),
            pltpu.VMEM_SHARED((_N, _D), jnp.float32),
            pltpu.SemaphoreType.DMA,
            pltpu.SemaphoreType.DMA,
            pltpu.SemaphoreType.DMA,
            pltpu.SemaphoreType.DMA,
            pltpu.SemaphoreType.DMA,
            pltpu.SemaphoreType.DMA,
            pltpu.SemaphoreType.DMA,
            pltpu.SemaphoreType.DMA,
            pltpu.SemaphoreType.DMA,
            pltpu.SemaphoreType.DMA,
        ],
    )
    def seg_kernel(x_hbm, e_hbm, w_hbm, z_hbm, out_hbm,
                   ib0, ib1, ib2, wb0, wb1, rows0, rows1, rout0, acc,
                   si0, si1, si2, sw0, sw1, sr0, sr1, ss0, ss1, sx0):
        cid = lax.axis_index("c")
        sid = lax.axis_index("s")
        wid = sid * _NC + cid
        r0 = pl.multiple_of(sid * _RPT, 8)

        pltpu.sync_copy(z_hbm, acc.at[pl.ds(r0, _RPT)])

        @pl.when(sid == 0)
        def _zrem():
            pltpu.sync_copy(z_hbm.at[pl.ds(0, _REM)],
                            acc.at[pl.ds(_REM0, _REM)])

        plsc.subcore_barrier()

        def idx_start(i, ib, sem):
            pltpu.async_copy(e_hbm.at[wid, i], ib, sem)

        def idx_drain(ib, sem):
            pltpu.make_async_copy(e_hbm.at[0, 0], ib, sem).wait()

        def w_start(i, wb, sem):
            pltpu.async_copy(w_hbm.at[wid, pl.ds(i, 1)], wb, sem)

        def w_drain(wb, sem):
            pltpu.make_async_copy(w_hbm.at[0, pl.ds(0, 1)], wb, sem).wait()

        def rows_start(ib, rows, sem):
            pltpu.async_copy(x_hbm.at[ib.at[0]], rows, sem)

        def rows_drain(rows, sem):
            pltpu.make_async_copy(x_hbm.at[pl.ds(0, _K)], rows, sem).wait()

        def mul(wb, rows, rout):
            def wgroup(j, c2):
                wvec = wb[0, pl.ds(pl.multiple_of(j * _L, _L), _L)]
                for t in range(_L):
                    k = j * _L + t
                    wk = wvec[t]
                    for c in range(_D // _L):
                        sl = pl.ds(c * _L, _L)
                        rout[k, sl] = rows[k, sl] * wk
                return c2

            lax.fori_loop(0, _K // _L, wgroup, 0)

        def scat_start(ib, rout, sem):
            pltpu.async_copy(rout, acc.at[ib.at[1]], sem, add=True)

        ibs = (ib0, ib1, ib2)
        sis = (si0, si1, si2)
        wbs = (wb0, wb1)
        sws = (sw0, sw1)
        rbs = (rows0, rows1)
        srs = (sr0, sr1)
        ros = (rout0, rout0)
        sss = (ss0, ss1)

        def step(i, t):
            if t != _NCH - 1:
                idx_drain(ibs[(t + 1) % 3], sis[(t + 1) % 3])
                rows_start(ibs[(t + 1) % 3], rbs[(t + 1) % 2], srs[(t + 1) % 2])
                w_start(i + 1, wbs[(t + 1) % 2], sws[(t + 1) % 2])
            rows_drain(rbs[t % 2], srs[t % 2])
            w_drain(wbs[t % 2], sws[t % 2])
            if t >= 1:
                rows_drain(rout0, sss[(t + 1) % 2])
            mul(wbs[t % 2], rbs[t % 2], rout0)
            scat_start(ibs[t % 3], rout0, sss[t % 2])
            if t < _NCH - 2:
                idx_start(i + 2, ibs[(t + 2) % 3], sis[(t + 2) % 3])

        pltpu.sync_copy(e_hbm.at[wid, 0], ib0)
        rows_start(ib0, rows0, sr0)
        w_start(0, wb0, sw0)
        idx_start(1, ib1, si1)

        step(0, 0)
        step(1, 1)

        def six(j, carry):
            i0 = 2 + j * 6
            for t in range(6):
                step(i0 + t, 2 + t)
            return carry

        lax.fori_loop(0, 20, six, 0)
        for t in range(122, _NCH):
            step(t, t)

        rows_drain(rout0, sss[(_NCH - 1) % 2])

        plsc.subcore_barrier()
        pltpu.sync_copy(acc.at[pl.ds(r0, _RPT)],
                        out_hbm.at[cid, pl.ds(r0, _RPT)])

        @pl.when(sid == 0)
        def _orem():
            pltpu.sync_copy(acc.at[pl.ds(_REM0, _REM)],
                            out_hbm.at[cid, pl.ds(_REM0, _REM)])

    return seg_kernel(x, edges3, w3, zeros_tile)


def _conv_dense(agg2, xin, W_rel, W_root, b, relu):
    R = 1000
    nblk = _N // R

    def body(agg_ref, x_ref, wr_ref, wt_ref, b_ref, o_ref):
        h = jnp.dot(agg_ref[0] + agg_ref[1], wr_ref[...],
                    preferred_element_type=jnp.float32)
        h = h + jnp.dot(x_ref[...], wt_ref[...],
                        preferred_element_type=jnp.float32)
        h = h + b_ref[...]
        if relu:
            h = jnp.maximum(h, 0.0)
        o_ref[...] = h

    return pl.pallas_call(
        body,
        grid=(nblk,),
        in_specs=[
            pl.BlockSpec((_NC, R, _D), lambda i: (0, i, 0)),
            pl.BlockSpec((R, _D), lambda i: (i, 0)),
            pl.BlockSpec((_D, _H), lambda i: (0, 0)),
            pl.BlockSpec((_D, _H), lambda i: (0, 0)),
            pl.BlockSpec((1, _H), lambda i: (0, 0)),
        ],
        out_specs=pl.BlockSpec((R, _H), lambda i: (i, 0)),
        out_shape=jax.ShapeDtypeStruct((_N, _H), jnp.float32),
    )(agg2, xin, W_rel, W_root, b.reshape(1, _H))


def _conv_pool_head(agg2, h1, W_rel, W_root, b, batch2d, Wfc_p, bfc_p):
    R = 1000
    nblk = _N // R

    def body(agg_ref, h_ref, wr_ref, wt_ref, b_ref, bt_ref, wfc_ref, bfc_ref,
             o_ref, sums, cnts):
        i = pl.program_id(0)

        @pl.when(i == 0)
        def _init():
            sums[...] = jnp.zeros_like(sums)
            cnts[...] = jnp.zeros_like(cnts)

        h = jnp.dot(agg_ref[0] + agg_ref[1], wr_ref[...],
                    preferred_element_type=jnp.float32)
        h = h + jnp.dot(h_ref[...], wt_ref[...],
                        preferred_element_type=jnp.float32)
        h = h + b_ref[...]
        onehot = (bt_ref[...] == lax.broadcasted_iota(jnp.int32, (1, _G), 1)
                  ).astype(jnp.float32)
        dims = (((0,), (0,)), ((), ()))
        sums[...] += lax.dot_general(onehot, h, dims,
                                     precision=lax.Precision.HIGHEST,
                                     preferred_element_type=jnp.float32)
        cnts[...] += lax.dot_general(onehot, jnp.ones_like(h), dims,
                                     precision=lax.Precision.HIGHEST,
                                     preferred_element_type=jnp.float32)

        @pl.when(i == nblk - 1)
        def _fin():
            pooled = sums[...] / jnp.maximum(cnts[...], 1.0)
            logits = jnp.dot(pooled, wfc_ref[...],
                             preferred_element_type=jnp.float32) + bfc_ref[...]
            o_ref[...] = jax.nn.sigmoid(logits)

    return pl.pallas_call(
        body,
        grid=(nblk,),
        in_specs=[
            pl.BlockSpec((_NC, R, _D), lambda i: (0, i, 0)),
            pl.BlockSpec((R, _H), lambda i: (i, 0)),
            pl.BlockSpec((_H, _H), lambda i: (0, 0)),
            pl.BlockSpec((_H, _H), lambda i: (0, 0)),
            pl.BlockSpec((1, _H), lambda i: (0, 0)),
            pl.BlockSpec((R, 1), lambda i: (i, 0)),
            pl.BlockSpec((_H, 128), lambda i: (0, 0)),
            pl.BlockSpec((1, 128), lambda i: (0, 0)),
        ],
        out_specs=pl.BlockSpec((_G, 128), lambda i: (0, 0)),
        out_shape=jax.ShapeDtypeStruct((_G, 128), jnp.float32),
        scratch_shapes=[pltpu.VMEM((_G, _H), jnp.float32),
                        pltpu.VMEM((_G, _H), jnp.float32)],
    )(agg2, h1, W_rel, W_root, b.reshape(1, _H), batch2d, Wfc_p, bfc_p)


def kernel(x, edge_index, edge_attr, batch,
           W1_rel, b1, W1_root, W2_rel, b2, W2_root, Wfc, bfc):
    src3 = edge_index[0].reshape(_NW, _NCH, _K)
    dst3 = edge_index[1].reshape(_NW, _NCH, _K)
    edges3 = jnp.stack([src3, dst3], axis=2)
    w3 = edge_attr.reshape(_NW, _NCH, _K)
    zeros_tile = jnp.zeros((_RPT, _D), jnp.float32)

    agg1 = _weighted_segsum(x, edges3, w3, zeros_tile)
    h1 = _conv_dense(agg1, x, W1_rel, W1_root, b1, relu=True)
    agg2 = _weighted_segsum(h1, edges3, w3, zeros_tile)

    batch2d = batch.reshape(_N, 1)
    Wfc_p = jnp.zeros((_H, 128), jnp.float32).at[:, :_C].set(Wfc)
    bfc_p = jnp.zeros((1, 128), jnp.float32).at[0, :_C].set(bfc)
    out = _conv_pool_head(agg2, h1, W2_rel, W2_root, b2,
                          batch2d, Wfc_p, bfc_p)
    return out[:, :_C]


# speedup vs baseline: 1.3465x; 1.3465x over previous
import functools

import jax
import jax.numpy as jnp
from jax import lax
from jax.experimental import pallas as pl
from jax.experimental.pallas import tpu as pltpu
from jax.experimental.pallas import tpu_sc as plsc

_N = 10000
_E = 320000
_D = 128
_H = 128
_C = 2
_G = 64

_NC = 2
_NS = 16
_L = 16
_NW = _NC * _NS
_EPW = _E // _NW
_K = 80
_NCH = _EPW // _K
_RPT = 624
_REM = _N - _NS * _RPT
_REM0 = _NS * _RPT


def _weighted_segsum(x, edges3, w3, zeros_tile):
    mesh = plsc.VectorSubcoreMesh(core_axis_name="c", subcore_axis_name="s")

    @functools.partial(
        pl.kernel,
        mesh=mesh,
        out_type=jax.ShapeDtypeStruct((_NC, _N, _D), jnp.float32),
        scratch_types=[
            pltpu.VMEM((2, _K), jnp.int32),
            pltpu.VMEM((2, _K), jnp.int32),
            pltpu.VMEM((2, _K), jnp.int32),
            pltpu.VMEM((1, _K), jnp.float32),
            pltpu.VMEM((1, _K), jnp.float32),
            pltpu.VMEM((_K, _D), jnp.float32),
            pltpu.VMEM((_K, _D), jnp.float32),
            pltpu.VMEM_SHARED((_N, _D), jnp.float32),
            pltpu.SemaphoreType.DMA,
            pltpu.SemaphoreType.DMA,
            pltpu.SemaphoreType.DMA,
            pltpu.SemaphoreType.DMA,
            pltpu.SemaphoreType.DMA,
            pltpu.SemaphoreType.DMA,
            pltpu.SemaphoreType.DMA,
            pltpu.SemaphoreType.DMA,
            pltpu.SemaphoreType.DMA,
        ],
    )
    def seg_kernel(x_hbm, e_hbm, w_hbm, z_hbm, out_hbm,
                   ib0, ib1, ib2, wb0, wb1, rows0, rows1, acc,
                   si0, si1, si2, sw0, sw1, sr0, sr1, ss0, ss1):
        cid = lax.axis_index("c")
        sid = lax.axis_index("s")
        wid = sid * _NC + cid
        r0 = pl.multiple_of(sid * _RPT, 8)

        pltpu.sync_copy(z_hbm, acc.at[pl.ds(r0, _RPT)])

        @pl.when(sid == 0)
        def _zrem():
            pltpu.sync_copy(z_hbm.at[pl.ds(0, _REM)],
                            acc.at[pl.ds(_REM0, _REM)])

        plsc.subcore_barrier()

        def idx_start(i, ib, sem):
            pltpu.async_copy(e_hbm.at[wid, i], ib, sem)

        def idx_drain(ib, sem):
            pltpu.make_async_copy(e_hbm.at[0, 0], ib, sem).wait()

        def w_start(i, wb, sem):
            pltpu.async_copy(w_hbm.at[wid, pl.ds(i, 1)], wb, sem)

        def w_drain(wb, sem):
            pltpu.make_async_copy(w_hbm.at[0, pl.ds(0, 1)], wb, sem).wait()

        def rows_start(ib, rows, sem):
            pltpu.async_copy(x_hbm.at[ib.at[0]], rows, sem)

        def rows_drain(rows, sem):
            pltpu.make_async_copy(x_hbm.at[pl.ds(0, _K)], rows, sem).wait()

        def mul(wb, rows):
            def wgroup(j, c2):
                wvec = wb[0, pl.ds(pl.multiple_of(j * _L, _L), _L)]
                for t in range(_L):
                    k = j * _L + t
                    wk = wvec[t]
                    for c in range(_D // _L):
                        sl = pl.ds(c * _L, _L)
                        rows[k, sl] = rows[k, sl] * wk
                return c2

            lax.fori_loop(0, _K // _L, wgroup, 0)

        def scat_start(ib, rout, sem):
            pltpu.async_copy(rout, acc.at[ib.at[1]], sem, add=True)

        ibs = (ib0, ib1, ib2)
        sis = (si0, si1, si2)
        wbs = (wb0, wb1)
        sws = (sw0, sw1)
        rbs = (rows0, rows1)
        srs = (sr0, sr1)
        sss = (ss0, ss1)

        def step(i, t):
            if t != _NCH - 1:
                idx_drain(ibs[(t + 1) % 3], sis[(t + 1) % 3])
                if t >= 1:
                    rows_drain(rbs[(t + 1) % 2], sss[(t + 1) % 2])
                rows_start(ibs[(t + 1) % 3], rbs[(t + 1) % 2], srs[(t + 1) % 2])
                w_start(i + 1, wbs[(t + 1) % 2], sws[(t + 1) % 2])
            if t < _NCH - 2:
                idx_start(i + 2, ibs[(t + 2) % 3], sis[(t + 2) % 3])
            rows_drain(rbs[t % 2], srs[t % 2])
            w_drain(wbs[t % 2], sws[t % 2])
            mul(wbs[t % 2], rbs[t % 2])
            scat_start(ibs[t % 3], rbs[t % 2], sss[t % 2])

        pltpu.sync_copy(e_hbm.at[wid, 0], ib0)
        rows_start(ib0, rows0, sr0)
        w_start(0, wb0, sw0)
        idx_start(1, ib1, si1)

        step(0, 0)
        step(1, 1)

        def six(j, carry):
            i0 = 2 + j * 6
            for t in range(6):
                step(i0 + t, 2 + t)
            return carry

        lax.fori_loop(0, 20, six, 0)
        for t in range(122, _NCH):
            step(t, t)

        rows_drain(rbs[1], sss[1])
        rows_drain(rbs[0], sss[0])

        plsc.subcore_barrier()
        pltpu.sync_copy(acc.at[pl.ds(r0, _RPT)],
                        out_hbm.at[cid, pl.ds(r0, _RPT)])

        @pl.when(sid == 0)
        def _orem():
            pltpu.sync_copy(acc.at[pl.ds(_REM0, _REM)],
                            out_hbm.at[cid, pl.ds(_REM0, _REM)])

    return seg_kernel(x, edges3, w3, zeros_tile)


def _conv_dense(agg2, xin, W_rel, W_root, b, relu):
    R = 1000
    nblk = _N // R

    def body(agg_ref, x_ref, wr_ref, wt_ref, b_ref, o_ref):
        h = jnp.dot(agg_ref[0] + agg_ref[1], wr_ref[...],
                    preferred_element_type=jnp.float32)
        h = h + jnp.dot(x_ref[...], wt_ref[...],
                        preferred_element_type=jnp.float32)
        h = h + b_ref[...]
        if relu:
            h = jnp.maximum(h, 0.0)
        o_ref[...] = h

    return pl.pallas_call(
        body,
        grid=(nblk,),
        in_specs=[
            pl.BlockSpec((_NC, R, _D), lambda i: (0, i, 0)),
            pl.BlockSpec((R, _D), lambda i: (i, 0)),
            pl.BlockSpec((_D, _H), lambda i: (0, 0)),
            pl.BlockSpec((_D, _H), lambda i: (0, 0)),
            pl.BlockSpec((1, _H), lambda i: (0, 0)),
        ],
        out_specs=pl.BlockSpec((R, _H), lambda i: (i, 0)),
        out_shape=jax.ShapeDtypeStruct((_N, _H), jnp.float32),
    )(agg2, xin, W_rel, W_root, b.reshape(1, _H))


def _conv_pool_head(agg2, h1, W_rel, W_root, b, batch2d, Wfc_p, bfc_p):
    R = 1000
    nblk = _N // R

    def body(agg_ref, h_ref, wr_ref, wt_ref, b_ref, bt_ref, wfc_ref, bfc_ref,
             o_ref, sums, cnts):
        i = pl.program_id(0)

        @pl.when(i == 0)
        def _init():
            sums[...] = jnp.zeros_like(sums)
            cnts[...] = jnp.zeros_like(cnts)

        h = jnp.dot(agg_ref[0] + agg_ref[1], wr_ref[...],
                    preferred_element_type=jnp.float32)
        h = h + jnp.dot(h_ref[...], wt_ref[...],
                        preferred_element_type=jnp.float32)
        h = h + b_ref[...]
        onehot = (bt_ref[...] == lax.broadcasted_iota(jnp.int32, (1, _G), 1)
                  ).astype(jnp.float32)
        dims = (((0,), (0,)), ((), ()))
        sums[...] += lax.dot_general(onehot, h, dims,
                                     precision=lax.Precision.HIGHEST,
                                     preferred_element_type=jnp.float32)
        cnts[...] += lax.dot_general(onehot, jnp.ones_like(h), dims,
                                     precision=lax.Precision.HIGHEST,
                                     preferred_element_type=jnp.float32)

        @pl.when(i == nblk - 1)
        def _fin():
            pooled = sums[...] / jnp.maximum(cnts[...], 1.0)
            logits = jnp.dot(pooled, wfc_ref[...],
                             preferred_element_type=jnp.float32) + bfc_ref[...]
            o_ref[...] = jax.nn.sigmoid(logits)

    return pl.pallas_call(
        body,
        grid=(nblk,),
        in_specs=[
            pl.BlockSpec((_NC, R, _D), lambda i: (0, i, 0)),
            pl.BlockSpec((R, _H), lambda i: (i, 0)),
            pl.BlockSpec((_H, _H), lambda i: (0, 0)),
            pl.BlockSpec((_H, _H), lambda i: (0, 0)),
            pl.BlockSpec((1, _H), lambda i: (0, 0)),
            pl.BlockSpec((R, 1), lambda i: (i, 0)),
            pl.BlockSpec((_H, 128), lambda i: (0, 0)),
            pl.BlockSpec((1, 128), lambda i: (0, 0)),
        ],
        out_specs=pl.BlockSpec((_G, 128), lambda i: (0, 0)),
        out_shape=jax.ShapeDtypeStruct((_G, 128), jnp.float32),
        scratch_shapes=[pltpu.VMEM((_G, _H), jnp.float32),
                        pltpu.VMEM((_G, _H), jnp.float32)],
    )(agg2, h1, W_rel, W_root, b.reshape(1, _H), batch2d, Wfc_p, bfc_p)


def kernel(x, edge_index, edge_attr, batch,
           W1_rel, b1, W1_root, W2_rel, b2, W2_root, Wfc, bfc):
    src3 = edge_index[0].reshape(_NW, _NCH, _K)
    dst3 = edge_index[1].reshape(_NW, _NCH, _K)
    edges3 = jnp.stack([src3, dst3], axis=2)
    w3 = edge_attr.reshape(_NW, _NCH, _K)
    zeros_tile = jnp.zeros((_RPT, _D), jnp.float32)

    agg1 = _weighted_segsum(x, edges3, w3, zeros_tile)
    h1 = _conv_dense(agg1, x, W1_rel, W1_root, b1, relu=True)
    agg2 = _weighted_segsum(h1, edges3, w3, zeros_tile)

    batch2d = batch.reshape(_N, 1)
    Wfc_p = jnp.zeros((_H, 128), jnp.float32).at[:, :_C].set(Wfc)
    bfc_p = jnp.zeros((1, 128), jnp.float32).at[0, :_C].set(bfc)
    out = _conv_pool_head(agg2, h1, W2_rel, W2_root, b2,
                          batch2d, Wfc_p, bfc_p)
    return out[:, :_C]
